# Initial kernel scaffold; baseline (speedup 1.0000x reference)
import jax
import jax.numpy as jnp
from jax.experimental import pallas as pl


def kernel(inputs, W):
    raise NotImplementedError("write your pallas kernel here")


# speedup vs baseline: 237.5897x; 237.5897x over previous
import functools

import jax
import jax.numpy as jnp
from jax import lax
from jax.experimental import pallas as pl
from jax.experimental.pallas import tpu as pltpu
from jax.experimental.pallas import tpu_sc as plsc

MAX_RANKS = 200
BATCH = 16384
LIST_LEN = 200

NUM_CORES = 2
NUM_SUBCORES = 16
LANES = 16

NW = NUM_CORES * NUM_SUBCORES
N_TOTAL = BATCH * LIST_LEN
N_PER = N_TOTAL // NW
CHUNK = 12800
N_CHUNKS = N_PER // CHUNK
VECS = CHUNK // LANES

_mesh = plsc.VectorSubcoreMesh(
    core_axis_name="c",
    subcore_axis_name="s",
    num_cores=NUM_CORES,
    num_subcores=NUM_SUBCORES,
)


@functools.partial(
    pl.kernel,
    out_type=jax.ShapeDtypeStruct((N_TOTAL,), jnp.float32),
    mesh=_mesh,
    compiler_params=pltpu.CompilerParams(needs_layout_passes=False),
    scratch_types=[
        pltpu.VMEM((MAX_RANKS,), jnp.float32),
        pltpu.VMEM((CHUNK,), jnp.int32),
        pltpu.VMEM((CHUNK,), jnp.float32),
    ],
)
def _positional_bias_kernel(idx_hbm, w_hbm, out_hbm, w_v, idx_v, out_v):
    wid = lax.axis_index("s") * NUM_CORES + lax.axis_index("c")
    base = wid * N_PER

    pltpu.sync_copy(w_hbm, w_v)

    for c in range(N_CHUNKS):
        off = base + c * CHUNK
        pltpu.sync_copy(idx_hbm.at[pl.ds(off, CHUNK)], idx_v)

        def body(i, _):
            raw = idx_v[pl.ds(i * LANES, LANES)]
            im1 = raw - 1
            valid = im1 >= 0
            safe = lax.max(im1, 0)
            g = plsc.load_gather(w_v, [safe])
            out_v[pl.ds(i * LANES, LANES)] = jnp.where(valid, g, 0.0)
            return 0

        lax.fori_loop(0, VECS, body, 0)

        pltpu.sync_copy(out_v, out_hbm.at[pl.ds(off, CHUNK)])


def kernel(inputs, W):
    out = _positional_bias_kernel(inputs.reshape(-1), W.reshape(-1))
    return out.reshape(BATCH, LIST_LEN, 1)


# speedup vs baseline: 304.3741x; 1.2811x over previous
import functools

import jax
import jax.numpy as jnp
from jax import lax
from jax.experimental import pallas as pl
from jax.experimental.pallas import tpu as pltpu
from jax.experimental.pallas import tpu_sc as plsc

MAX_RANKS = 200
BATCH = 16384
LIST_LEN = 200

NUM_CORES = 2
NUM_SUBCORES = 16
LANES = 16

NW = NUM_CORES * NUM_SUBCORES
N_TOTAL = BATCH * LIST_LEN
N_PER = N_TOTAL // NW
CHUNK = 12800
N_CHUNKS = N_PER // CHUNK

_mesh = plsc.VectorSubcoreMesh(
    core_axis_name="c",
    subcore_axis_name="s",
    num_cores=NUM_CORES,
    num_subcores=NUM_SUBCORES,
)


@functools.partial(
    pl.kernel,
    out_type=jax.ShapeDtypeStruct((N_TOTAL,), jnp.float32),
    mesh=_mesh,
    compiler_params=pltpu.CompilerParams(needs_layout_passes=False),
    scratch_types=[
        pltpu.VMEM((MAX_RANKS,), jnp.float32),
        pltpu.VMEM((CHUNK,), jnp.int32),
        pltpu.VMEM((CHUNK,), jnp.int32),
        pltpu.VMEM((CHUNK,), jnp.float32),
        pltpu.VMEM((CHUNK,), jnp.float32),
        pltpu.SemaphoreType.DMA,
        pltpu.SemaphoreType.DMA,
    ],
)
def _positional_bias_kernel(
    idx_hbm, w_hbm, out_hbm, w_v, idx_v0, idx_v1, out_v0, out_v1,
    in_sem, out_sem,
):
    wid = lax.axis_index("s") * NUM_CORES + lax.axis_index("c")
    base = wid * N_PER

    pltpu.sync_copy(w_hbm, w_v)

    idx_bufs = (idx_v0, idx_v1)
    out_bufs = (out_v0, out_v1)

    in_copies = [None] * N_CHUNKS
    out_copies = [None] * N_CHUNKS

    in_copies[0] = pltpu.async_copy(
        idx_hbm.at[pl.ds(base, CHUNK)], idx_bufs[0], in_sem)

    for c in range(N_CHUNKS):
        idx_v = idx_bufs[c % 2]
        out_v = out_bufs[c % 2]

        if c + 1 < N_CHUNKS:
            in_copies[c + 1] = pltpu.async_copy(
                idx_hbm.at[pl.ds(base + (c + 1) * CHUNK, CHUNK)],
                idx_bufs[(c + 1) % 2], in_sem)

        in_copies[c].wait()
        if c >= 2:
            out_copies[c - 2].wait()

        @plsc.parallel_loop(0, CHUNK, step=LANES, unroll=8)
        def _gather_body(i):
            raw = idx_v[pl.ds(i, LANES)]
            im1 = raw - 1
            valid = im1 >= 0
            safe = lax.max(im1, 0)
            g = plsc.load_gather(w_v, [safe])
            out_v[pl.ds(i, LANES)] = jnp.where(valid, g, 0.0)

        out_copies[c] = pltpu.async_copy(
            out_v, out_hbm.at[pl.ds(base + c * CHUNK, CHUNK)], out_sem)

    out_copies[N_CHUNKS - 2].wait()
    out_copies[N_CHUNKS - 1].wait()


def kernel(inputs, W):
    out = _positional_bias_kernel(inputs.reshape(-1), W.reshape(-1))
    return out.reshape(BATCH, LIST_LEN, 1)


# speedup vs baseline: 504.3004x; 1.6568x over previous
import functools

import jax
import jax.numpy as jnp
from jax import lax
from jax.experimental import pallas as pl
from jax.experimental.pallas import tpu as pltpu
from jax.experimental.pallas import tpu_sc as plsc

MAX_RANKS = 200
BATCH = 16384
LIST_LEN = 200

NUM_CORES = 2
NUM_SUBCORES = 16
LANES = 16

NW = NUM_CORES * NUM_SUBCORES
ROWS_PER = BATCH // NW
ROW_CHUNK = 64
N_CHUNKS = ROWS_PER // ROW_CHUNK

COL_OFFS = tuple(range(0, LIST_LEN - LANES + 1, LANES)) + (LIST_LEN - LANES,)

_mesh = plsc.VectorSubcoreMesh(
    core_axis_name="c",
    subcore_axis_name="s",
    num_cores=NUM_CORES,
    num_subcores=NUM_SUBCORES,
)


@functools.partial(
    pl.kernel,
    out_type=jax.ShapeDtypeStruct((BATCH, LIST_LEN), jnp.float32),
    mesh=_mesh,
    compiler_params=pltpu.CompilerParams(needs_layout_passes=False),
    scratch_types=[
        pltpu.VMEM((MAX_RANKS,), jnp.float32),
        pltpu.VMEM((ROW_CHUNK, LIST_LEN), jnp.int32),
        pltpu.VMEM((ROW_CHUNK, LIST_LEN), jnp.int32),
        pltpu.VMEM((ROW_CHUNK, LIST_LEN), jnp.float32),
        pltpu.VMEM((ROW_CHUNK, LIST_LEN), jnp.float32),
        pltpu.SemaphoreType.DMA,
        pltpu.SemaphoreType.DMA,
    ],
)
def _positional_bias_kernel(
    idx_hbm, w_hbm, out_hbm, w_v, idx_v0, idx_v1, out_v0, out_v1,
    in_sem, out_sem,
):
    wid = lax.axis_index("s") * NUM_CORES + lax.axis_index("c")
    base = wid * ROWS_PER

    pltpu.sync_copy(w_hbm, w_v)

    idx_bufs = (idx_v0, idx_v1)
    out_bufs = (out_v0, out_v1)

    in_copies = [None] * N_CHUNKS
    out_copies = [None] * N_CHUNKS

    in_copies[0] = pltpu.async_copy(
        idx_hbm.at[pl.ds(base, ROW_CHUNK)], idx_bufs[0], in_sem)

    for c in range(N_CHUNKS):
        idx_v = idx_bufs[c % 2]
        out_v = out_bufs[c % 2]

        if c + 1 < N_CHUNKS:
            in_copies[c + 1] = pltpu.async_copy(
                idx_hbm.at[pl.ds(base + (c + 1) * ROW_CHUNK, ROW_CHUNK)],
                idx_bufs[(c + 1) % 2], in_sem)

        in_copies[c].wait()
        if c >= 2:
            out_copies[c - 2].wait()

        @plsc.parallel_loop(0, ROW_CHUNK, step=1, unroll=2)
        def _gather_body(r):
            for off in COL_OFFS:
                raw = idx_v[r, pl.ds(off, LANES)]
                im1 = raw - 1
                valid = im1 >= 0
                safe = lax.max(im1, 0)
                g = plsc.load_gather(w_v, [safe])
                out_v[r, pl.ds(off, LANES)] = jnp.where(valid, g, 0.0)

        out_copies[c] = pltpu.async_copy(
            out_v, out_hbm.at[pl.ds(base + c * ROW_CHUNK, ROW_CHUNK)], out_sem)

    out_copies[N_CHUNKS - 2].wait()
    out_copies[N_CHUNKS - 1].wait()


def kernel(inputs, W):
    out = _positional_bias_kernel(inputs, W.reshape(-1))
    return out[:, :, None]


# speedup vs baseline: 551.6243x; 1.0938x over previous
import functools

import jax
import jax.numpy as jnp
from jax import lax
from jax.experimental import pallas as pl
from jax.experimental.pallas import tpu as pltpu
from jax.experimental.pallas import tpu_sc as plsc

MAX_RANKS = 200
BATCH = 16384
LIST_LEN = 200

NUM_CORES = 2
NUM_SUBCORES = 16
LANES = 16

NW = NUM_CORES * NUM_SUBCORES
COLS_PER = BATCH // NW
ROW_CHUNK = 40
N_CHUNKS = LIST_LEN // ROW_CHUNK
COL_VECS = COLS_PER // LANES

_mesh = plsc.VectorSubcoreMesh(
    core_axis_name="c",
    subcore_axis_name="s",
    num_cores=NUM_CORES,
    num_subcores=NUM_SUBCORES,
)


@functools.partial(
    pl.kernel,
    out_type=jax.ShapeDtypeStruct((LIST_LEN, BATCH), jnp.float32),
    mesh=_mesh,
    compiler_params=pltpu.CompilerParams(needs_layout_passes=False),
    scratch_types=[
        pltpu.VMEM((MAX_RANKS,), jnp.float32),
        pltpu.VMEM((ROW_CHUNK, COLS_PER), jnp.int32),
        pltpu.VMEM((ROW_CHUNK, COLS_PER), jnp.int32),
        pltpu.VMEM((ROW_CHUNK, COLS_PER), jnp.float32),
        pltpu.VMEM((ROW_CHUNK, COLS_PER), jnp.float32),
        pltpu.SemaphoreType.DMA,
        pltpu.SemaphoreType.DMA,
    ],
)
def _positional_bias_kernel(
    idx_hbm, w_hbm, out_hbm, w_v, idx_v0, idx_v1, out_v0, out_v1,
    in_sem, out_sem,
):
    wid = lax.axis_index("s") * NUM_CORES + lax.axis_index("c")
    col0 = wid * COLS_PER

    pltpu.sync_copy(w_hbm, w_v)

    idx_bufs = (idx_v0, idx_v1)
    out_bufs = (out_v0, out_v1)

    in_copies = [None] * N_CHUNKS
    out_copies = [None] * N_CHUNKS

    in_copies[0] = pltpu.async_copy(
        idx_hbm.at[pl.ds(0, ROW_CHUNK), pl.ds(col0, COLS_PER)],
        idx_bufs[0], in_sem)

    for c in range(N_CHUNKS):
        idx_v = idx_bufs[c % 2]
        out_v = out_bufs[c % 2]

        if c + 1 < N_CHUNKS:
            in_copies[c + 1] = pltpu.async_copy(
                idx_hbm.at[pl.ds((c + 1) * ROW_CHUNK, ROW_CHUNK),
                           pl.ds(col0, COLS_PER)],
                idx_bufs[(c + 1) % 2], in_sem)

        in_copies[c].wait()
        if c >= 2:
            out_copies[c - 2].wait()

        @plsc.parallel_loop(0, ROW_CHUNK, step=1, unroll=2)
        def _gather_body(r):
            for cc in range(COL_VECS):
                raw = idx_v[r, pl.ds(cc * LANES, LANES)]
                im1 = raw - 1
                valid = im1 >= 0
                safe = lax.max(im1, 0)
                g = plsc.load_gather(w_v, [safe])
                out_v[r, pl.ds(cc * LANES, LANES)] = jnp.where(valid, g, 0.0)

        out_copies[c] = pltpu.async_copy(
            out_v,
            out_hbm.at[pl.ds(c * ROW_CHUNK, ROW_CHUNK), pl.ds(col0, COLS_PER)],
            out_sem)

    out_copies[N_CHUNKS - 2].wait()
    out_copies[N_CHUNKS - 1].wait()


def kernel(inputs, W):
    out_t = _positional_bias_kernel(inputs.T, W.reshape(-1))
    return out_t.T[:, :, None]


# speedup vs baseline: 702.3407x; 1.2732x over previous
import functools

import jax
import jax.numpy as jnp
from jax import lax
from jax.experimental import pallas as pl
from jax.experimental.pallas import tpu as pltpu
from jax.experimental.pallas import tpu_sc as plsc

MAX_RANKS = 200
BATCH = 16384
LIST_LEN = 200

NUM_CORES = 2
NUM_SUBCORES = 16
LANES = 16

NW = NUM_CORES * NUM_SUBCORES
COLS_PER = BATCH // NW
ROW_CHUNK = 40
N_CHUNKS = LIST_LEN // ROW_CHUNK
COL_VECS = COLS_PER // LANES

_mesh = plsc.VectorSubcoreMesh(
    core_axis_name="c",
    subcore_axis_name="s",
    num_cores=NUM_CORES,
    num_subcores=NUM_SUBCORES,
)


@functools.partial(
    pl.kernel,
    out_type=jax.ShapeDtypeStruct((BATCH * LIST_LEN,), jnp.float32),
    mesh=_mesh,
    compiler_params=pltpu.CompilerParams(needs_layout_passes=False),
    scratch_types=[
        pltpu.VMEM((MAX_RANKS,), jnp.float32),
        pltpu.VMEM((ROW_CHUNK, COLS_PER), jnp.int32),
        pltpu.VMEM((ROW_CHUNK, COLS_PER), jnp.int32),
        pltpu.VMEM((ROW_CHUNK, COLS_PER), jnp.float32),
        pltpu.VMEM((ROW_CHUNK, COLS_PER), jnp.float32),
        pltpu.SemaphoreType.DMA,
        pltpu.SemaphoreType.DMA,
    ],
)
def _positional_bias_kernel(
    idx_hbm, t_hbm, out_hbm, t_v, idx_v0, idx_v1, out_v0, out_v1,
    in_sem, out_sem,
):
    wid = lax.axis_index("s") * NUM_CORES + lax.axis_index("c")
    col0 = wid * COLS_PER

    pltpu.sync_copy(t_hbm, t_v)

    idx_bufs = (idx_v0, idx_v1)
    out_bufs = (out_v0, out_v1)

    in_copies = [None] * N_CHUNKS
    out_copies = [[] for _ in range(N_CHUNKS)]

    in_copies[0] = pltpu.async_copy(
        idx_hbm.at[pl.ds(0, ROW_CHUNK), pl.ds(col0, COLS_PER)],
        idx_bufs[0], in_sem)

    for c in range(N_CHUNKS):
        idx_v = idx_bufs[c % 2]
        out_v = out_bufs[c % 2]

        if c + 1 < N_CHUNKS:
            in_copies[c + 1] = pltpu.async_copy(
                idx_hbm.at[pl.ds((c + 1) * ROW_CHUNK, ROW_CHUNK),
                           pl.ds(col0, COLS_PER)],
                idx_bufs[(c + 1) % 2], in_sem)

        in_copies[c].wait()
        if c >= 2:
            for cp in out_copies[c - 2]:
                cp.wait()

        @plsc.parallel_loop(0, ROW_CHUNK, step=1, unroll=2)
        def _gather_body(r):
            for cc in range(COL_VECS):
                raw = idx_v[r, pl.ds(cc * LANES, LANES)]
                out_v[r, pl.ds(cc * LANES, LANES)] = plsc.load_gather(
                    t_v, [raw])

        for r in range(ROW_CHUNK):
            out_copies[c].append(pltpu.async_copy(
                out_v.at[r],
                out_hbm.at[pl.ds((c * ROW_CHUNK + r) * BATCH + col0,
                                 COLS_PER)],
                out_sem))

    for c in (N_CHUNKS - 2, N_CHUNKS - 1):
        for cp in out_copies[c]:
            cp.wait()


def kernel(inputs, W):
    table = jnp.concatenate([jnp.zeros((1,), jnp.float32), W[:MAX_RANKS - 1, 0]])
    flat = _positional_bias_kernel(inputs.T, table)
    return flat.reshape(1, LIST_LEN, BATCH).transpose(2, 1, 0)


# speedup vs baseline: 779.6545x; 1.1101x over previous
import functools

import jax
import jax.numpy as jnp
from jax import lax
from jax.experimental import pallas as pl
from jax.experimental.pallas import tpu as pltpu
from jax.experimental.pallas import tpu_sc as plsc

MAX_RANKS = 200
BATCH = 16384
LIST_LEN = 200

NUM_CORES = 2
NUM_SUBCORES = 16
LANES = 16

NW = NUM_CORES * NUM_SUBCORES
COLS_PER = BATCH // NW
ROW_CHUNK = 40
N_CHUNKS = LIST_LEN // ROW_CHUNK
COL_VECS = COLS_PER // LANES

_mesh = plsc.VectorSubcoreMesh(
    core_axis_name="c",
    subcore_axis_name="s",
    num_cores=NUM_CORES,
    num_subcores=NUM_SUBCORES,
)


@functools.partial(
    pl.kernel,
    out_type=jax.ShapeDtypeStruct((LIST_LEN, 1, BATCH), jnp.float32),
    mesh=_mesh,
    compiler_params=pltpu.CompilerParams(needs_layout_passes=False),
    scratch_types=[
        pltpu.VMEM((MAX_RANKS,), jnp.float32),
        pltpu.VMEM((ROW_CHUNK, COLS_PER), jnp.int32),
        pltpu.VMEM((ROW_CHUNK, COLS_PER), jnp.int32),
        pltpu.VMEM((ROW_CHUNK, COLS_PER), jnp.float32),
        pltpu.VMEM((ROW_CHUNK, COLS_PER), jnp.float32),
        pltpu.SemaphoreType.DMA,
        pltpu.SemaphoreType.DMA,
    ],
)
def _positional_bias_kernel(
    idx_hbm, t_hbm, out_hbm, t_v, idx_v0, idx_v1, out_v0, out_v1,
    in_sem, out_sem,
):
    wid = lax.axis_index("s") * NUM_CORES + lax.axis_index("c")
    col0 = wid * COLS_PER

    pltpu.sync_copy(t_hbm, t_v)

    idx_bufs = (idx_v0, idx_v1)
    out_bufs = (out_v0, out_v1)

    in_copies = [None] * N_CHUNKS
    out_copies = [None] * N_CHUNKS

    in_copies[0] = pltpu.async_copy(
        idx_hbm.at[pl.ds(0, ROW_CHUNK), pl.ds(col0, COLS_PER)],
        idx_bufs[0], in_sem)

    for c in range(N_CHUNKS):
        idx_v = idx_bufs[c % 2]
        out_v = out_bufs[c % 2]

        if c + 1 < N_CHUNKS:
            in_copies[c + 1] = pltpu.async_copy(
                idx_hbm.at[pl.ds((c + 1) * ROW_CHUNK, ROW_CHUNK),
                           pl.ds(col0, COLS_PER)],
                idx_bufs[(c + 1) % 2], in_sem)

        in_copies[c].wait()
        if c >= 2:
            out_copies[c - 2].wait()

        @plsc.parallel_loop(0, ROW_CHUNK, step=1, unroll=4)
        def _gather_body(r):
            for cc in range(COL_VECS):
                raw = idx_v[r, pl.ds(cc * LANES, LANES)]
                out_v[r, pl.ds(cc * LANES, LANES)] = plsc.load_gather(
                    t_v, [raw])

        out_copies[c] = pltpu.async_copy(
            out_v,
            out_hbm.at[pl.ds(c * ROW_CHUNK, ROW_CHUNK), 0,
                       pl.ds(col0, COLS_PER)],
            out_sem)

    out_copies[N_CHUNKS - 2].wait()
    out_copies[N_CHUNKS - 1].wait()


def kernel(inputs, W):
    table = jnp.concatenate([jnp.zeros((1,), jnp.float32), W[:MAX_RANKS - 1, 0]])
    out3 = _positional_bias_kernel(inputs.T, table)
    return out3.transpose(2, 0, 1)


# speedup vs baseline: 787.0137x; 1.0094x over previous
import functools

import jax
import jax.numpy as jnp
from jax import lax
from jax.experimental import pallas as pl
from jax.experimental.pallas import tpu as pltpu
from jax.experimental.pallas import tpu_sc as plsc

MAX_RANKS = 200
BATCH = 16384
LIST_LEN = 200

NUM_CORES = 2
NUM_SUBCORES = 16
LANES = 16

NW = NUM_CORES * NUM_SUBCORES
COLS_PER = BATCH // NW
ROW_CHUNK = 40
N_CHUNKS = LIST_LEN // ROW_CHUNK
COL_VECS = COLS_PER // LANES

_mesh = plsc.VectorSubcoreMesh(
    core_axis_name="c",
    subcore_axis_name="s",
    num_cores=NUM_CORES,
    num_subcores=NUM_SUBCORES,
)


@functools.partial(
    pl.kernel,
    out_type=jax.ShapeDtypeStruct((LIST_LEN, 1, BATCH), jnp.float32),
    mesh=_mesh,
    compiler_params=pltpu.CompilerParams(needs_layout_passes=False),
    scratch_types=[
        pltpu.VMEM((MAX_RANKS,), jnp.float32),
        pltpu.VMEM((ROW_CHUNK, COLS_PER), jnp.int32),
        pltpu.VMEM((ROW_CHUNK, COLS_PER), jnp.int32),
        pltpu.VMEM((ROW_CHUNK, COLS_PER), jnp.float32),
        pltpu.VMEM((ROW_CHUNK, COLS_PER), jnp.float32),
        pltpu.SemaphoreType.DMA,
        pltpu.SemaphoreType.DMA,
    ],
)
def _positional_bias_kernel(
    idx_hbm, t_hbm, out_hbm, t_v, idx_v0, idx_v1, out_v0, out_v1,
    in_sem, out_sem,
):
    wid = lax.axis_index("s") * NUM_CORES + lax.axis_index("c")
    col0 = wid * COLS_PER

    pltpu.sync_copy(t_hbm, t_v)

    idx_bufs = (idx_v0, idx_v1)
    out_bufs = (out_v0, out_v1)

    in_copies = [None] * N_CHUNKS
    out_copies = [None] * N_CHUNKS

    in_copies[0] = pltpu.async_copy(
        idx_hbm.at[pl.ds(0, ROW_CHUNK), pl.ds(col0, COLS_PER)],
        idx_bufs[0], in_sem)

    for c in range(N_CHUNKS):
        idx_v = idx_bufs[c % 2]
        out_v = out_bufs[c % 2]

        if c + 1 < N_CHUNKS:
            in_copies[c + 1] = pltpu.async_copy(
                idx_hbm.at[pl.ds((c + 1) * ROW_CHUNK, ROW_CHUNK),
                           pl.ds(col0, COLS_PER)],
                idx_bufs[(c + 1) % 2], in_sem)

        in_copies[c].wait()
        if c >= 2:
            out_copies[c - 2].wait()

        @plsc.parallel_loop(0, ROW_CHUNK, step=1, unroll=8)
        def _gather_body(r):
            for cc in range(COL_VECS):
                raw = idx_v[r, pl.ds(cc * LANES, LANES)]
                out_v[r, pl.ds(cc * LANES, LANES)] = plsc.load_gather(
                    t_v, [raw])

        out_copies[c] = pltpu.async_copy(
            out_v,
            out_hbm.at[pl.ds(c * ROW_CHUNK, ROW_CHUNK), 0,
                       pl.ds(col0, COLS_PER)],
            out_sem)

    out_copies[N_CHUNKS - 2].wait()
    out_copies[N_CHUNKS - 1].wait()


def kernel(inputs, W):
    table = jnp.concatenate([jnp.zeros((1,), jnp.float32), W[:MAX_RANKS - 1, 0]])
    out3 = _positional_bias_kernel(inputs.T, table)
    return out3.transpose(2, 0, 1)


# speedup vs baseline: 830.2967x; 1.0550x over previous
import functools

import jax
import jax.numpy as jnp
from jax import lax
from jax.experimental import pallas as pl
from jax.experimental.pallas import tpu as pltpu
from jax.experimental.pallas import tpu_sc as plsc

MAX_RANKS = 200
BATCH = 16384
LIST_LEN = 200

NUM_CORES = 2
NUM_SUBCORES = 16
LANES = 16

NW = NUM_CORES * NUM_SUBCORES
COLS_PER = BATCH // NW
ROW_CHUNK = 40
N_CHUNKS = LIST_LEN // ROW_CHUNK
COL_VECS = COLS_PER // LANES

_mesh = plsc.VectorSubcoreMesh(
    core_axis_name="c",
    subcore_axis_name="s",
    num_cores=NUM_CORES,
    num_subcores=NUM_SUBCORES,
)


@functools.partial(
    pl.kernel,
    out_type=jax.ShapeDtypeStruct((LIST_LEN, 1, BATCH), jnp.float32),
    mesh=_mesh,
    compiler_params=pltpu.CompilerParams(needs_layout_passes=False),
    scratch_types=[
        pltpu.VMEM((MAX_RANKS + 8,), jnp.float32),
        pltpu.VMEM((MAX_RANKS,), jnp.float32),
        pltpu.VMEM((ROW_CHUNK, COLS_PER), jnp.int32),
        pltpu.VMEM((ROW_CHUNK, COLS_PER), jnp.int32),
        pltpu.VMEM((ROW_CHUNK, COLS_PER), jnp.float32),
        pltpu.VMEM((ROW_CHUNK, COLS_PER), jnp.float32),
        pltpu.SemaphoreType.DMA,
        pltpu.SemaphoreType.DMA,
    ],
)
def _positional_bias_kernel(
    idx_hbm, w_hbm, out_hbm, t_v, w_v, idx_v0, idx_v1, out_v0, out_v1,
    in_sem, out_sem,
):
    wid = lax.axis_index("s") * NUM_CORES + lax.axis_index("c")
    col0 = wid * COLS_PER

    pltpu.sync_copy(w_hbm, w_v)
    for k in range(0, MAX_RANKS, LANES):
        ii = lax.iota(jnp.int32, LANES) + (k - 1)
        safe = jnp.clip(ii, 0, MAX_RANKS - 1)
        g = plsc.load_gather(w_v, [safe])
        t_v[pl.ds(k, LANES)] = jnp.where(ii >= 0, g, 0.0)

    idx_bufs = (idx_v0, idx_v1)
    out_bufs = (out_v0, out_v1)

    in_copies = [None] * N_CHUNKS
    out_copies = [None] * N_CHUNKS

    in_copies[0] = pltpu.async_copy(
        idx_hbm.at[pl.ds(0, ROW_CHUNK), pl.ds(col0, COLS_PER)],
        idx_bufs[0], in_sem)

    for c in range(N_CHUNKS):
        idx_v = idx_bufs[c % 2]
        out_v = out_bufs[c % 2]

        if c + 1 < N_CHUNKS:
            in_copies[c + 1] = pltpu.async_copy(
                idx_hbm.at[pl.ds((c + 1) * ROW_CHUNK, ROW_CHUNK),
                           pl.ds(col0, COLS_PER)],
                idx_bufs[(c + 1) % 2], in_sem)

        in_copies[c].wait()
        if c >= 2:
            out_copies[c - 2].wait()

        @plsc.parallel_loop(0, ROW_CHUNK, step=1, unroll=8)
        def _gather_body(r):
            for cc in range(COL_VECS):
                raw = idx_v[r, pl.ds(cc * LANES, LANES)]
                out_v[r, pl.ds(cc * LANES, LANES)] = plsc.load_gather(
                    t_v, [raw])

        out_copies[c] = pltpu.async_copy(
            out_v,
            out_hbm.at[pl.ds(c * ROW_CHUNK, ROW_CHUNK), 0,
                       pl.ds(col0, COLS_PER)],
            out_sem)

    out_copies[N_CHUNKS - 2].wait()
    out_copies[N_CHUNKS - 1].wait()


def kernel(inputs, W):
    out3 = _positional_bias_kernel(inputs.T, W.reshape(-1))
    return out3.transpose(2, 0, 1)
